# Initial kernel scaffold; baseline (speedup 1.0000x reference)
import jax
import jax.numpy as jnp
from jax.experimental import pallas as pl


def kernel(x, edge_index, W_pre, b_pre, W_u1, b_u1, W_u2, b_u2):
    raise NotImplementedError("write your pallas kernel here")


# speedup vs baseline: 6.4981x; 6.4981x over previous
import functools

import jax
import jax.numpy as jnp
from jax import lax
from jax.experimental import pallas as pl
from jax.experimental.pallas import tpu as pltpu
from jax.experimental.pallas import tpu_sc as plsc

N = 10000
E = 320000
D = 128

NC = 2
NS = 16
NW = NC * NS
CHUNK = 128
NCHUNKS = E // CHUNK
BASE_CHUNKS = NCHUNKS // NW
EXTRA = NCHUNKS - BASE_CHUNKS * NW
R_STD = 624
R_LAST = N - (NS - 1) * R_STD


def _pre_body(x_ref, w_ref, b_ref, o_ref):
    o_ref[...] = jnp.maximum(
        jnp.dot(x_ref[...], w_ref[...], preferred_element_type=jnp.float32)
        + b_ref[...], 0.0)


def _update_body(x_ref, z0_ref, z1_ref, w1x_ref, w1z_ref, b1_ref, w2_ref,
                 b2_ref, o_ref):
    z = z0_ref[...] + z1_ref[...]
    t = jnp.maximum(
        jnp.dot(x_ref[...], w1x_ref[...], preferred_element_type=jnp.float32)
        + jnp.dot(z, w1z_ref[...], preferred_element_type=jnp.float32)
        + b1_ref[...], 0.0)
    o_ref[...] = (jnp.dot(t, w2_ref[...], preferred_element_type=jnp.float32)
                  + b2_ref[...])


_mesh = plsc.VectorSubcoreMesh(core_axis_name="c", subcore_axis_name="s")


@functools.partial(
    pl.kernel,
    out_type=jax.ShapeDtypeStruct((NC, N, D), jnp.float32),
    mesh=_mesh,
    scratch_types=[
        pltpu.VMEM((CHUNK,), jnp.int32),
        pltpu.VMEM((CHUNK,), jnp.int32),
        pltpu.VMEM((CHUNK, D), jnp.float32),
        pltpu.VMEM_SHARED((N, D), jnp.float32),
        pltpu.SemaphoreType.DMA,
    ],
)
def _segment_sum_sc(m_hbm, src_hbm, dst_hbm, zeros_hbm, out_hbm,
                    src_v, dst_v, rows_v, z_sh, sem):
    cid = lax.axis_index("c")
    sid = lax.axis_index("s")
    wid = sid * NC + cid

    r0 = pl.multiple_of(sid * R_STD, 8)

    @pl.when(sid < NS - 1)
    def _():
        pltpu.sync_copy(zeros_hbm.at[pl.ds(0, R_STD)],
                        z_sh.at[pl.ds(r0, R_STD)])

    @pl.when(sid == NS - 1)
    def _():
        pltpu.sync_copy(zeros_hbm, z_sh.at[pl.ds(r0, R_LAST)])

    plsc.subcore_barrier()

    def do_chunk(c):
        base = c * CHUNK
        pltpu.sync_copy(src_hbm.at[pl.ds(base, CHUNK)], src_v)
        pltpu.sync_copy(dst_hbm.at[pl.ds(base, CHUNK)], dst_v)
        pltpu.async_copy(m_hbm.at[src_v], rows_v, sem).wait()
        pltpu.sync_copy(rows_v, z_sh.at[dst_v], add=True)

    def body(i, carry):
        do_chunk(wid * BASE_CHUNKS + i)
        return carry

    lax.fori_loop(0, BASE_CHUNKS, body, 0)

    @pl.when(wid < EXTRA)
    def _():
        do_chunk(NW * BASE_CHUNKS + wid)

    plsc.subcore_barrier()

    @pl.when(sid < NS - 1)
    def _():
        pltpu.sync_copy(z_sh.at[pl.ds(r0, R_STD)],
                        out_hbm.at[cid, pl.ds(r0, R_STD)])

    @pl.when(sid == NS - 1)
    def _():
        pltpu.sync_copy(z_sh.at[pl.ds(r0, R_LAST)],
                        out_hbm.at[cid, pl.ds(r0, R_LAST)])


def kernel(x, edge_index, W_pre, b_pre, W_u1, b_u1, W_u2, b_u2):
    src = edge_index[0].astype(jnp.int32)
    dst = edge_index[1].astype(jnp.int32)

    m = pl.pallas_call(
        _pre_body,
        out_shape=jax.ShapeDtypeStruct((N, D), jnp.float32),
    )(x, W_pre, b_pre.reshape(1, D))

    zeros = jnp.zeros((R_LAST, D), dtype=jnp.float32)
    z_parts = _segment_sum_sc(m, src, dst, zeros)

    h = pl.pallas_call(
        _update_body,
        out_shape=jax.ShapeDtypeStruct((N, D), jnp.float32),
    )(x, z_parts[0], z_parts[1], W_u1[:D], W_u1[D:], b_u1.reshape(1, D),
      W_u2, b_u2.reshape(1, D))
    return h
